# Initial kernel scaffold; baseline (speedup 1.0000x reference)
import jax
import jax.numpy as jnp
from jax.experimental import pallas as pl


def kernel(p0, x0, o0, n0, params):
    raise NotImplementedError("write your pallas kernel here")


# speedup vs baseline: 1.0045x; 1.0045x over previous
import jax
import jax.numpy as jnp
import numpy as np
from jax.experimental import pallas as pl

N_POINTS = 10000
BLOCKS = [2, 3, 4, 6, 3]
PLANES = [32, 64, 128, 256, 512]
HIDDEN = [min(p, 256) for p in PLANES]
STRIDE = [1, 4, 4, 4, 4]
NSAMPLE = [36, 24, 24, 24, 24]
NUM_HEADS = 4
C_IN = 1
MLP_DIMS = [512, 512, 1024, 1024, 512, 256, 128, 64, 32, 3]


def _knn_idx(query_p, ref_p, k):
    q = jax.lax.stop_gradient(query_p)
    r = jax.lax.stop_gradient(ref_p)
    d = jnp.sum(q * q, -1)[:, None] - 2.0 * (q @ r.T) + jnp.sum(r * r, -1)[None, :]
    _, idx = jax.lax.top_k(-d, k)
    return idx


def _fps_idx(p, m):
    p = jax.lax.stop_gradient(p)
    N = p.shape[0]
    def body(i, state):
        dists, idxs = state
        last = p[idxs[i - 1]]
        d = jnp.sum((p - last[None, :]) ** 2, -1)
        dists = jnp.minimum(dists, d)
        idxs = idxs.at[i].set(jnp.argmax(dists).astype(jnp.int32))
        return dists, idxs
    dists = jnp.full((N,), jnp.inf, jnp.float32)
    idxs = jnp.zeros((m,), jnp.int32)
    _, idxs = jax.lax.fori_loop(1, m, body, (dists, idxs))
    return idxs


def _calc_ppf(p, n, p_r, n_r):
    d = p_r - p[:, None, :]
    dist = jnp.linalg.norm(d, axis=-1, keepdims=True)
    dn = d / jnp.maximum(dist, 1e-8)
    def angle(a, b):
        cr = jnp.cross(a, b)
        return jnp.arctan2(jnp.linalg.norm(cr, axis=-1), jnp.sum(a * b, -1))
    n1 = jnp.broadcast_to(n[:, None, :], p_r.shape)
    a1 = angle(n1, dn)
    a2 = angle(n_r, dn)
    a3 = angle(n1, n_r)
    return jnp.stack([a1, a2, a3, dist[..., 0]], axis=-1)


def _local_ppf_transformer(tp, x, node_idx, group_idx, ppf):
    q = x[node_idx] @ tp["q"]["w"] + tp["q"]["b"]
    xg = x[group_idx]
    k = xg @ tp["k"]["w"] + tp["k"]["b"]
    v = xg @ tp["v"]["w"] + tp["v"]["b"]
    pe = jax.nn.relu(ppf @ tp["p1"]["w"] + tp["p1"]["b"]) @ tp["p2"]["w"] + tp["p2"]["b"]
    M, K, h = k.shape
    dh = h // NUM_HEADS
    qh = q.reshape(M, NUM_HEADS, dh)
    kh = (k + pe).reshape(M, K, NUM_HEADS, dh)
    vh = (v + pe).reshape(M, K, NUM_HEADS, dh)
    attn = jnp.einsum('mhd,mkhd->mkh', qh, kh) / np.sqrt(dh)
    attn = jax.nn.softmax(attn, axis=1)
    out = jnp.einsum('mkh,mkhd->mhd', attn, vh).reshape(M, h)
    return out @ tp["o"]["w"] + tp["o"]["b"]


def _layer_norm(x, g, b):
    mu = jnp.mean(x, -1, keepdims=True)
    var = jnp.var(x, -1, keepdims=True)
    return (x - mu) / jnp.sqrt(var + 1e-5) * g + b


def _mlp_kernel_body(x_ref, *refs):
    out_ref = refs[-1]
    wrefs = refs[:-1]
    x = x_ref[...]
    nl = len(wrefs) // 2
    for i in range(nl):
        w = wrefs[2 * i][...]
        b = wrefs[2 * i + 1][...]
        x = jnp.dot(x, w, preferred_element_type=jnp.float32) + b[None, :]
        x = jnp.tanh(x) if i == nl - 1 else jax.nn.relu(x)
    out_ref[...] = x


def _mlp_pallas(x, mlp_params):
    M = x.shape[0]
    args = [x]
    for lp in mlp_params:
        args.append(lp["w"])
        args.append(lp["b"])
    out = pl.pallas_call(
        _mlp_kernel_body,
        out_shape=jax.ShapeDtypeStruct((M, MLP_DIMS[-1]), jnp.float32),
    )(*args)
    return out


def _forward(params, p, x, n):
    for s in range(5):
        stage = params["stages"][s]
        stride, nsample = STRIDE[s], NSAMPLE[s]
        if stride != 1:
            m = p.shape[0] // stride
            idx = _fps_idx(p, m)
            n_p = p[idx]
            n_n = n[idx]
        else:
            idx = jnp.arange(p.shape[0], dtype=jnp.int32)
            n_p = p
            n_n = n
        group_idx = _knn_idx(n_p, p, nsample)
        ppf = _calc_ppf(n_p, n_n, p[group_idx], n[group_idx])
        x = _local_ppf_transformer(stage["down"], x, idx, group_idx, ppf)
        p, n = n_p, n_n
        if len(stage["blocks"]) > 0:
            bg_idx = _knn_idx(p, p, nsample)
            node_idx = jnp.arange(p.shape[0], dtype=jnp.int32)
            bppf = _calc_ppf(p, n, p[bg_idx], n[bg_idx])
            for blk in stage["blocks"]:
                identity = x
                xt = _local_ppf_transformer(blk["tr"], x, node_idx, bg_idx, bppf)
                xt = _layer_norm(xt, blk["ln_g"], blk["ln_b"])
                x = jax.nn.relu(xt + identity)
    return _mlp_pallas(x, params["mlp"])


def kernel(p0, x0, o0, n0, params):
    return _forward(params, p0, x0, n0)


# speedup vs baseline: 5.1968x; 5.1736x over previous
import jax
import jax.numpy as jnp
import numpy as np
from jax.experimental import pallas as pl

N_POINTS = 10000
BLOCKS = [2, 3, 4, 6, 3]
PLANES = [32, 64, 128, 256, 512]
HIDDEN = [min(p, 256) for p in PLANES]
STRIDE = [1, 4, 4, 4, 4]
NSAMPLE = [36, 24, 24, 24, 24]
NUM_HEADS = 4
C_IN = 1
MLP_DIMS = [512, 512, 1024, 1024, 512, 256, 128, 64, 32, 3]


def _pad128(n):
    return max(128, ((n + 127) // 128) * 128)


def _knn_kernel_body(q_ref, rt_ref, rsq_ref, out_ref, *, nsample, bits):
    lmask = (1 << bits) - 1
    q = q_ref[...]
    qsq = jnp.sum(q * q, axis=1, keepdims=True)
    d = qsq - 2.0 * jnp.dot(q, rt_ref[...], preferred_element_type=jnp.float32)
    d = d + rsq_ref[...]
    d = jnp.maximum(d, 0.0)
    di = jax.lax.bitcast_convert_type(d, jnp.int32)
    lane = jax.lax.broadcasted_iota(jnp.int32, d.shape, 1)
    key = (di & ~lmask) | lane
    prev = jnp.full((q.shape[0], 1), -1, jnp.int32)
    imax = jnp.int32(2147483647)
    cols = []
    for _ in range(nsample):
        cand = jnp.where(key > prev, key, imax)
        m = jnp.min(cand, axis=1, keepdims=True)
        cols.append(m & lmask)
        prev = m
    out_ref[...] = jnp.concatenate(cols, axis=1)


def _knn_idx(query_p, ref_p, k):
    M = query_p.shape[0]
    N = ref_p.shape[0]
    Npad = _pad128(N)
    bits = int(np.ceil(np.log2(Npad)))
    BM = min(128, _pad128(M))
    Mpad = ((M + BM - 1) // BM) * BM
    q = jnp.pad(query_p, ((0, Mpad - M), (0, 0)))
    rt = jnp.pad(ref_p.T, ((0, 0), (0, Npad - N)))
    rsq = jnp.pad(jnp.sum(ref_p * ref_p, -1), (0, Npad - N),
                  constant_values=1e30)[None, :]
    import functools
    body = functools.partial(_knn_kernel_body, nsample=k, bits=bits)
    out = pl.pallas_call(
        body,
        grid=(Mpad // BM,),
        in_specs=[
            pl.BlockSpec((BM, 3), lambda i: (i, 0)),
            pl.BlockSpec((3, Npad), lambda i: (0, 0)),
            pl.BlockSpec((1, Npad), lambda i: (0, 0)),
        ],
        out_specs=pl.BlockSpec((BM, k), lambda i: (i, 0)),
        out_shape=jax.ShapeDtypeStruct((Mpad, k), jnp.int32),
    )(q, rt, rsq)
    return out[:M]


def _fps_kernel_body(pt_ref, out_ref, *, N, m, Npad, mpad):
    x = pt_ref[0:1, :]
    y = pt_ref[1:2, :]
    z = pt_ref[2:3, :]
    lane = jax.lax.broadcasted_iota(jnp.int32, (1, Npad), 1)
    lane_m = jax.lax.broadcasted_iota(jnp.int32, (1, mpad), 1)
    dists0 = jnp.where(lane < N, jnp.inf, -1.0)
    idxvec0 = jnp.zeros((1, mpad), jnp.int32)
    big = jnp.int32(2147483647)

    def body(i, carry):
        dists, idxvec, lx, ly, lz = carry
        dx = x - lx
        dy = y - ly
        dz = z - lz
        d = dx * dx + dy * dy + dz * dz
        dists = jnp.minimum(dists, d)
        mval = jnp.max(dists)
        eq = dists == mval
        idx = jnp.min(jnp.where(eq, lane, big))
        eq2 = lane == idx
        lx = jnp.max(jnp.where(eq2, x, -jnp.inf))
        ly = jnp.max(jnp.where(eq2, y, -jnp.inf))
        lz = jnp.max(jnp.where(eq2, z, -jnp.inf))
        idxvec = jnp.where(lane_m == i, idx, idxvec)
        return (dists, idxvec, lx, ly, lz)

    lx0 = pt_ref[0, 0]
    ly0 = pt_ref[1, 0]
    lz0 = pt_ref[2, 0]
    carry = (dists0, idxvec0, lx0, ly0, lz0)
    carry = jax.lax.fori_loop(1, m, body, carry)
    out_ref[...] = carry[1]


def _fps_idx(p, m):
    N = p.shape[0]
    Npad = _pad128(N)
    mpad = _pad128(m)
    pt = jnp.pad(p.T, ((0, 0), (0, Npad - N)))
    import functools
    body = functools.partial(_fps_kernel_body, N=N, m=m, Npad=Npad, mpad=mpad)
    out = pl.pallas_call(
        body,
        out_shape=jax.ShapeDtypeStruct((1, mpad), jnp.int32),
    )(pt)
    return out[0, :m]


def _calc_ppf(p, n, p_r, n_r):
    d = p_r - p[:, None, :]
    dist = jnp.linalg.norm(d, axis=-1, keepdims=True)
    dn = d / jnp.maximum(dist, 1e-8)
    def angle(a, b):
        cr = jnp.cross(a, b)
        return jnp.arctan2(jnp.linalg.norm(cr, axis=-1), jnp.sum(a * b, -1))
    n1 = jnp.broadcast_to(n[:, None, :], p_r.shape)
    a1 = angle(n1, dn)
    a2 = angle(n_r, dn)
    a3 = angle(n1, n_r)
    return jnp.stack([a1, a2, a3, dist[..., 0]], axis=-1)


def _local_ppf_transformer(tp, x, node_idx, group_idx, ppf):
    q = x[node_idx] @ tp["q"]["w"] + tp["q"]["b"]
    xg = x[group_idx]
    k = xg @ tp["k"]["w"] + tp["k"]["b"]
    v = xg @ tp["v"]["w"] + tp["v"]["b"]
    pe = jax.nn.relu(ppf @ tp["p1"]["w"] + tp["p1"]["b"]) @ tp["p2"]["w"] + tp["p2"]["b"]
    M, K, h = k.shape
    dh = h // NUM_HEADS
    qh = q.reshape(M, NUM_HEADS, dh)
    kh = (k + pe).reshape(M, K, NUM_HEADS, dh)
    vh = (v + pe).reshape(M, K, NUM_HEADS, dh)
    attn = jnp.einsum('mhd,mkhd->mkh', qh, kh) / np.sqrt(dh)
    attn = jax.nn.softmax(attn, axis=1)
    out = jnp.einsum('mkh,mkhd->mhd', attn, vh).reshape(M, h)
    return out @ tp["o"]["w"] + tp["o"]["b"]


def _layer_norm(x, g, b):
    mu = jnp.mean(x, -1, keepdims=True)
    var = jnp.var(x, -1, keepdims=True)
    return (x - mu) / jnp.sqrt(var + 1e-5) * g + b


def _mlp_kernel_body(x_ref, *refs):
    out_ref = refs[-1]
    wrefs = refs[:-1]
    x = x_ref[...]
    nl = len(wrefs) // 2
    for i in range(nl):
        w = wrefs[2 * i][...]
        b = wrefs[2 * i + 1][...]
        x = jnp.dot(x, w, preferred_element_type=jnp.float32) + b[None, :]
        x = jnp.tanh(x) if i == nl - 1 else jax.nn.relu(x)
    out_ref[...] = x


def _mlp_pallas(x, mlp_params):
    M = x.shape[0]
    args = [x]
    for lp in mlp_params:
        args.append(lp["w"])
        args.append(lp["b"])
    out = pl.pallas_call(
        _mlp_kernel_body,
        out_shape=jax.ShapeDtypeStruct((M, MLP_DIMS[-1]), jnp.float32),
    )(*args)
    return out


def _forward(params, p, x, n):
    for s in range(5):
        stage = params["stages"][s]
        stride, nsample = STRIDE[s], NSAMPLE[s]
        if stride != 1:
            m = p.shape[0] // stride
            idx = _fps_idx(p, m)
            n_p = p[idx]
            n_n = n[idx]
        else:
            idx = jnp.arange(p.shape[0], dtype=jnp.int32)
            n_p = p
            n_n = n
        group_idx = _knn_idx(n_p, p, nsample)
        ppf = _calc_ppf(n_p, n_n, p[group_idx], n[group_idx])
        x = _local_ppf_transformer(stage["down"], x, idx, group_idx, ppf)
        p, n = n_p, n_n
        if len(stage["blocks"]) > 0:
            bg_idx = _knn_idx(p, p, nsample)
            node_idx = jnp.arange(p.shape[0], dtype=jnp.int32)
            bppf = _calc_ppf(p, n, p[bg_idx], n[bg_idx])
            for blk in stage["blocks"]:
                identity = x
                xt = _local_ppf_transformer(blk["tr"], x, node_idx, bg_idx, bppf)
                xt = _layer_norm(xt, blk["ln_g"], blk["ln_b"])
                x = jax.nn.relu(xt + identity)
    return _mlp_pallas(x, params["mlp"])


def kernel(p0, x0, o0, n0, params):
    return _forward(params, p0, x0, n0)


# speedup vs baseline: 6.1852x; 1.1902x over previous
import functools

import jax
import jax.numpy as jnp
import numpy as np
from jax import lax
from jax.experimental import pallas as pl
from jax.experimental.pallas import tpu as pltpu
from jax.experimental.pallas import tpu_sc as plsc

N_POINTS = 10000
BLOCKS = [2, 3, 4, 6, 3]
PLANES = [32, 64, 128, 256, 512]
HIDDEN = [min(p, 256) for p in PLANES]
STRIDE = [1, 4, 4, 4, 4]
NSAMPLE = [36, 24, 24, 24, 24]
NUM_HEADS = 4
C_IN = 1
MLP_DIMS = [512, 512, 1024, 1024, 512, 256, 128, 64, 32, 3]


def _pad128(n):
    return max(128, ((n + 127) // 128) * 128)


_SC_NC = 2
_SC_NS = 16
_SC_NW = _SC_NC * _SC_NS


def _sc_gather_kernel(table_hbm, idx_hbm, out_hbm, idx_v, rows_v, sem,
                      *, ch, nloop, b_per_w):
    wid = lax.axis_index("s") * _SC_NC + lax.axis_index("c")
    base = wid * b_per_w

    def chunk(i, carry):
        off = base + i * ch
        pltpu.sync_copy(idx_hbm.at[pl.ds(off, ch)], idx_v)
        pltpu.async_copy(table_hbm.at[idx_v], rows_v, sem).wait()
        pltpu.sync_copy(rows_v, out_hbm.at[pl.ds(off, ch)])
        return carry

    lax.fori_loop(0, nloop, chunk, 0)


def _sc_gather(table, idx):
    B = idx.shape[0]
    D = table.shape[1]
    ch_max = (393216 // ((D + 1) * 4)) // 8 * 8
    ch_max = min(2048, ch_max)
    need = -(-B // _SC_NW)
    if need <= ch_max:
        ch = -(-need // 8) * 8
        nloop = 1
    else:
        ch = ch_max
        nloop = -(-need // ch)
    b_per_w = ch * nloop
    Bpad = b_per_w * _SC_NW
    idx_p = jnp.pad(idx, (0, Bpad - B))
    mesh = plsc.VectorSubcoreMesh(core_axis_name="c", subcore_axis_name="s")
    body = functools.partial(_sc_gather_kernel, ch=ch, nloop=nloop,
                             b_per_w=b_per_w)
    run = pl.kernel(
        body,
        mesh=mesh,
        compiler_params=pltpu.CompilerParams(use_tc_tiling_on_sc=False),
        out_type=jax.ShapeDtypeStruct((Bpad, D), jnp.float32),
        scratch_types=[
            pltpu.VMEM((ch,), jnp.int32),
            pltpu.VMEM((ch, D), jnp.float32),
            pltpu.SemaphoreType.DMA,
        ],
    )
    out = run(table, idx_p)
    return out[:B]


def _knn_kernel_body(q_ref, rt_ref, rsq_ref, out_ref, *, nsample, bits):
    lmask = (1 << bits) - 1
    q = q_ref[...]
    qsq = jnp.sum(q * q, axis=1, keepdims=True)
    d = qsq - 2.0 * jnp.dot(q, rt_ref[...], preferred_element_type=jnp.float32)
    d = d + rsq_ref[...]
    d = jnp.maximum(d, 0.0)
    di = jax.lax.bitcast_convert_type(d, jnp.int32)
    lane = jax.lax.broadcasted_iota(jnp.int32, d.shape, 1)
    key = (di & ~lmask) | lane
    prev = jnp.full((q.shape[0], 1), -1, jnp.int32)
    imax = jnp.int32(2147483647)
    cols = []
    for _ in range(nsample):
        cand = jnp.where(key > prev, key, imax)
        m = jnp.min(cand, axis=1, keepdims=True)
        cols.append(m & lmask)
        prev = m
    out_ref[...] = jnp.concatenate(cols, axis=1)


def _knn_idx(query_p, ref_p, k):
    M = query_p.shape[0]
    N = ref_p.shape[0]
    Npad = _pad128(N)
    bits = int(np.ceil(np.log2(Npad)))
    BM = min(128, _pad128(M))
    Mpad = ((M + BM - 1) // BM) * BM
    q = jnp.pad(query_p, ((0, Mpad - M), (0, 0)))
    rt = jnp.pad(ref_p.T, ((0, 0), (0, Npad - N)))
    rsq = jnp.pad(jnp.sum(ref_p * ref_p, -1), (0, Npad - N),
                  constant_values=1e30)[None, :]
    import functools
    body = functools.partial(_knn_kernel_body, nsample=k, bits=bits)
    out = pl.pallas_call(
        body,
        grid=(Mpad // BM,),
        in_specs=[
            pl.BlockSpec((BM, 3), lambda i: (i, 0)),
            pl.BlockSpec((3, Npad), lambda i: (0, 0)),
            pl.BlockSpec((1, Npad), lambda i: (0, 0)),
        ],
        out_specs=pl.BlockSpec((BM, k), lambda i: (i, 0)),
        out_shape=jax.ShapeDtypeStruct((Mpad, k), jnp.int32),
    )(q, rt, rsq)
    return out[:M]


def _fps_kernel_body(pt_ref, out_ref, *, N, m, Npad, mpad):
    x = pt_ref[0:1, :]
    y = pt_ref[1:2, :]
    z = pt_ref[2:3, :]
    lane = jax.lax.broadcasted_iota(jnp.int32, (1, Npad), 1)
    lane_m = jax.lax.broadcasted_iota(jnp.int32, (1, mpad), 1)
    dists0 = jnp.where(lane < N, jnp.inf, -1.0)
    idxvec0 = jnp.zeros((1, mpad), jnp.int32)
    big = jnp.int32(2147483647)

    def body(i, carry):
        dists, idxvec, lx, ly, lz = carry
        dx = x - lx
        dy = y - ly
        dz = z - lz
        d = dx * dx + dy * dy + dz * dz
        dists = jnp.minimum(dists, d)
        mval = jnp.max(dists)
        eq = dists == mval
        idx = jnp.min(jnp.where(eq, lane, big))
        eq2 = lane == idx
        lx = jnp.max(jnp.where(eq2, x, -jnp.inf))
        ly = jnp.max(jnp.where(eq2, y, -jnp.inf))
        lz = jnp.max(jnp.where(eq2, z, -jnp.inf))
        idxvec = jnp.where(lane_m == i, idx, idxvec)
        return (dists, idxvec, lx, ly, lz)

    lx0 = pt_ref[0, 0]
    ly0 = pt_ref[1, 0]
    lz0 = pt_ref[2, 0]
    carry = (dists0, idxvec0, lx0, ly0, lz0)
    carry = jax.lax.fori_loop(1, m, body, carry)
    out_ref[...] = carry[1]


def _fps_idx(p, m):
    N = p.shape[0]
    Npad = _pad128(N)
    mpad = _pad128(m)
    pt = jnp.pad(p.T, ((0, 0), (0, Npad - N)))
    import functools
    body = functools.partial(_fps_kernel_body, N=N, m=m, Npad=Npad, mpad=mpad)
    out = pl.pallas_call(
        body,
        out_shape=jax.ShapeDtypeStruct((1, mpad), jnp.int32),
    )(pt)
    return out[0, :m]


def _calc_ppf(p, n, p_r, n_r):
    d = p_r - p[:, None, :]
    dist = jnp.linalg.norm(d, axis=-1, keepdims=True)
    dn = d / jnp.maximum(dist, 1e-8)
    def angle(a, b):
        cr = jnp.cross(a, b)
        return jnp.arctan2(jnp.linalg.norm(cr, axis=-1), jnp.sum(a * b, -1))
    n1 = jnp.broadcast_to(n[:, None, :], p_r.shape)
    a1 = angle(n1, dn)
    a2 = angle(n_r, dn)
    a3 = angle(n1, n_r)
    return jnp.stack([a1, a2, a3, dist[..., 0]], axis=-1)


def _gather_rows(x, idx_flat):
    D = x.shape[1]
    Dp = ((D + 15) // 16) * 16
    if Dp != D:
        x = jnp.pad(x, ((0, 0), (0, Dp - D)))
    return _sc_gather(x, idx_flat)[:, :D]


def _local_ppf_transformer(tp, xq_raw, xg, ppf):
    q = xq_raw @ tp["q"]["w"] + tp["q"]["b"]
    k = xg @ tp["k"]["w"] + tp["k"]["b"]
    v = xg @ tp["v"]["w"] + tp["v"]["b"]
    pe = jax.nn.relu(ppf @ tp["p1"]["w"] + tp["p1"]["b"]) @ tp["p2"]["w"] + tp["p2"]["b"]
    M, K, h = k.shape
    dh = h // NUM_HEADS
    qh = q.reshape(M, NUM_HEADS, dh)
    kh = (k + pe).reshape(M, K, NUM_HEADS, dh)
    vh = (v + pe).reshape(M, K, NUM_HEADS, dh)
    attn = jnp.einsum('mhd,mkhd->mkh', qh, kh) / np.sqrt(dh)
    attn = jax.nn.softmax(attn, axis=1)
    out = jnp.einsum('mkh,mkhd->mhd', attn, vh).reshape(M, h)
    return out @ tp["o"]["w"] + tp["o"]["b"]


def _layer_norm(x, g, b):
    mu = jnp.mean(x, -1, keepdims=True)
    var = jnp.var(x, -1, keepdims=True)
    return (x - mu) / jnp.sqrt(var + 1e-5) * g + b


def _mlp_kernel_body(x_ref, *refs):
    out_ref = refs[-1]
    wrefs = refs[:-1]
    x = x_ref[...]
    nl = len(wrefs) // 2
    for i in range(nl):
        w = wrefs[2 * i][...]
        b = wrefs[2 * i + 1][...]
        x = jnp.dot(x, w, preferred_element_type=jnp.float32) + b[None, :]
        x = jnp.tanh(x) if i == nl - 1 else jax.nn.relu(x)
    out_ref[...] = x


def _mlp_pallas(x, mlp_params):
    M = x.shape[0]
    args = [x]
    for lp in mlp_params:
        args.append(lp["w"])
        args.append(lp["b"])
    out = pl.pallas_call(
        _mlp_kernel_body,
        out_shape=jax.ShapeDtypeStruct((M, MLP_DIMS[-1]), jnp.float32),
    )(*args)
    return out


def _forward(params, p, x, n):
    for s in range(5):
        stage = params["stages"][s]
        stride, nsample = STRIDE[s], NSAMPLE[s]
        N = p.shape[0]
        pn = jnp.concatenate([p, n], axis=1)
        if stride != 1:
            m = N // stride
            idx = _fps_idx(p, m)
            pn_q = _gather_rows(pn, idx)
            n_p = pn_q[:, 0:3]
            n_n = pn_q[:, 3:6]
            xq = _gather_rows(x, idx)
        else:
            m = N
            n_p, n_n = p, n
            xq = x
        group_idx = _knn_idx(n_p, p, nsample)
        gg = _gather_rows(pn, group_idx.reshape(-1)).reshape(m, nsample, 6)
        ppf = _calc_ppf(n_p, n_n, gg[..., 0:3], gg[..., 3:6])
        xg = _gather_rows(x, group_idx.reshape(-1)).reshape(m, nsample, -1)
        x = _local_ppf_transformer(stage["down"], xq, xg, ppf)
        p, n = n_p, n_n
        if len(stage["blocks"]) > 0:
            pn = jnp.concatenate([p, n], axis=1)
            bg_idx = _knn_idx(p, p, nsample)
            bgf = bg_idx.reshape(-1)
            bgg = _gather_rows(pn, bgf).reshape(m, nsample, 6)
            bppf = _calc_ppf(p, n, bgg[..., 0:3], bgg[..., 3:6])
            for blk in stage["blocks"]:
                identity = x
                bxg = _gather_rows(x, bgf).reshape(m, nsample, -1)
                xt = _local_ppf_transformer(blk["tr"], x, bxg, bppf)
                xt = _layer_norm(xt, blk["ln_g"], blk["ln_b"])
                x = jax.nn.relu(xt + identity)
    return _mlp_pallas(x, params["mlp"])


def kernel(p0, x0, o0, n0, params):
    return _forward(params, p0, x0, n0)


# speedup vs baseline: 6.3988x; 1.0345x over previous
import functools

import jax
import jax.numpy as jnp
import numpy as np
from jax import lax
from jax.experimental import pallas as pl
from jax.experimental.pallas import tpu as pltpu
from jax.experimental.pallas import tpu_sc as plsc

N_POINTS = 10000
BLOCKS = [2, 3, 4, 6, 3]
PLANES = [32, 64, 128, 256, 512]
HIDDEN = [min(p, 256) for p in PLANES]
STRIDE = [1, 4, 4, 4, 4]
NSAMPLE = [36, 24, 24, 24, 24]
NUM_HEADS = 4
C_IN = 1
MLP_DIMS = [512, 512, 1024, 1024, 512, 256, 128, 64, 32, 3]


def _pad128(n):
    return max(128, ((n + 127) // 128) * 128)


_SC_NC = 2
_SC_NS = 16
_SC_NW = _SC_NC * _SC_NS


def _sc_gather_kernel(table_hbm, idx_hbm, out_hbm, idx_v, rows_v, sem,
                      *, ch, nloop, b_per_w):
    wid = lax.axis_index("s") * _SC_NC + lax.axis_index("c")
    base = wid * b_per_w

    def chunk(i, carry):
        off = base + i * ch
        pltpu.sync_copy(idx_hbm.at[pl.ds(off, ch)], idx_v)
        pltpu.async_copy(table_hbm.at[idx_v], rows_v, sem).wait()
        pltpu.sync_copy(rows_v, out_hbm.at[pl.ds(off, ch)])
        return carry

    lax.fori_loop(0, nloop, chunk, 0)


def _sc_gather(table, idx):
    B = idx.shape[0]
    D = table.shape[1]
    ch_max = (393216 // ((D + 1) * 4)) // 8 * 8
    ch_max = min(2048, ch_max)
    need = -(-B // _SC_NW)
    if need <= ch_max:
        ch = -(-need // 8) * 8
        nloop = 1
    else:
        ch = ch_max
        nloop = -(-need // ch)
    b_per_w = ch * nloop
    Bpad = b_per_w * _SC_NW
    idx_p = jnp.pad(idx, (0, Bpad - B))
    mesh = plsc.VectorSubcoreMesh(core_axis_name="c", subcore_axis_name="s")
    body = functools.partial(_sc_gather_kernel, ch=ch, nloop=nloop,
                             b_per_w=b_per_w)
    run = pl.kernel(
        body,
        mesh=mesh,
        compiler_params=pltpu.CompilerParams(use_tc_tiling_on_sc=False),
        out_type=jax.ShapeDtypeStruct((Bpad, D), jnp.float32),
        scratch_types=[
            pltpu.VMEM((ch,), jnp.int32),
            pltpu.VMEM((ch, D), jnp.float32),
            pltpu.SemaphoreType.DMA,
        ],
    )
    out = run(table, idx_p)
    return out[:B]


def _knn_kernel_body(q_ref, rt_ref, rsq_ref, out_ref, *, nsample, bits):
    lmask = (1 << bits) - 1
    q = q_ref[...]
    qsq = jnp.sum(q * q, axis=1, keepdims=True)
    d = qsq - 2.0 * jnp.dot(q, rt_ref[...], preferred_element_type=jnp.float32)
    d = d + rsq_ref[...]
    d = jnp.maximum(d, 0.0)
    di = jax.lax.bitcast_convert_type(d, jnp.int32)
    lane = jax.lax.broadcasted_iota(jnp.int32, d.shape, 1)
    key = (di & ~lmask) | lane
    prev = jnp.full((q.shape[0], 1), -1, jnp.int32)
    imax = jnp.int32(2147483647)
    cols = []
    for _ in range(nsample):
        cand = jnp.where(key > prev, key, imax)
        m = jnp.min(cand, axis=1, keepdims=True)
        cols.append(m & lmask)
        prev = m
    out_ref[...] = jnp.concatenate(cols, axis=1)


def _knn_idx(query_p, ref_p, k):
    M = query_p.shape[0]
    N = ref_p.shape[0]
    Npad = _pad128(N)
    bits = int(np.ceil(np.log2(Npad)))
    BM = min(128, _pad128(M))
    Mpad = ((M + BM - 1) // BM) * BM
    q = jnp.pad(query_p, ((0, Mpad - M), (0, 0)))
    rt = jnp.pad(ref_p.T, ((0, 0), (0, Npad - N)))
    rsq = jnp.pad(jnp.sum(ref_p * ref_p, -1), (0, Npad - N),
                  constant_values=1e30)[None, :]
    import functools
    body = functools.partial(_knn_kernel_body, nsample=k, bits=bits)
    out = pl.pallas_call(
        body,
        grid=(Mpad // BM,),
        in_specs=[
            pl.BlockSpec((BM, 3), lambda i: (i, 0)),
            pl.BlockSpec((3, Npad), lambda i: (0, 0)),
            pl.BlockSpec((1, Npad), lambda i: (0, 0)),
        ],
        out_specs=pl.BlockSpec((BM, k), lambda i: (i, 0)),
        out_shape=jax.ShapeDtypeStruct((Mpad, k), jnp.int32),
    )(q, rt, rsq)
    return out[:M]


def _fps_kernel_body(pt_ref, out_ref, *, N, m, Npad, mpad):
    x = pt_ref[0:1, :]
    y = pt_ref[1:2, :]
    z = pt_ref[2:3, :]
    lane = jax.lax.broadcasted_iota(jnp.int32, (1, Npad), 1)
    lane_m = jax.lax.broadcasted_iota(jnp.int32, (1, mpad), 1)
    dists0 = jnp.where(lane < N, jnp.inf, -1.0)
    idxvec0 = jnp.zeros((1, mpad), jnp.int32)
    big = jnp.int32(2147483647)

    def body(i, carry):
        dists, idxvec, lx, ly, lz = carry
        dx = x - lx
        dy = y - ly
        dz = z - lz
        d = dx * dx + dy * dy + dz * dz
        dists = jnp.minimum(dists, d)
        mval = jnp.max(dists)
        eq = dists == mval
        idx = jnp.min(jnp.where(eq, lane, big))
        eq2 = lane == idx
        lx = jnp.max(jnp.where(eq2, x, -jnp.inf))
        ly = jnp.max(jnp.where(eq2, y, -jnp.inf))
        lz = jnp.max(jnp.where(eq2, z, -jnp.inf))
        idxvec = jnp.where(lane_m == i, idx, idxvec)
        return (dists, idxvec, lx, ly, lz)

    lx0 = pt_ref[0, 0]
    ly0 = pt_ref[1, 0]
    lz0 = pt_ref[2, 0]
    carry = (dists0, idxvec0, lx0, ly0, lz0)
    carry = jax.lax.fori_loop(1, m, body, carry)
    out_ref[...] = carry[1]


def _fps_idx(p, m):
    N = p.shape[0]
    Npad = _pad128(N)
    mpad = _pad128(m)
    pt = jnp.pad(p.T, ((0, 0), (0, Npad - N)))
    import functools
    body = functools.partial(_fps_kernel_body, N=N, m=m, Npad=Npad, mpad=mpad)
    out = pl.pallas_call(
        body,
        out_shape=jax.ShapeDtypeStruct((1, mpad), jnp.int32),
    )(pt)
    return out[0, :m]


def _calc_ppf(p, n, p_r, n_r):
    d = p_r - p[:, None, :]
    dist = jnp.linalg.norm(d, axis=-1, keepdims=True)
    dn = d / jnp.maximum(dist, 1e-8)
    def angle(a, b):
        cr = jnp.cross(a, b)
        return jnp.arctan2(jnp.linalg.norm(cr, axis=-1), jnp.sum(a * b, -1))
    n1 = jnp.broadcast_to(n[:, None, :], p_r.shape)
    a1 = angle(n1, dn)
    a2 = angle(n_r, dn)
    a3 = angle(n1, n_r)
    return jnp.stack([a1, a2, a3, dist[..., 0]], axis=-1)


def _gather_rows(x, idx_flat):
    D = x.shape[1]
    Dp = ((D + 15) // 16) * 16
    if Dp != D:
        x = jnp.pad(x, ((0, 0), (0, Dp - D)))
    return _sc_gather(x, idx_flat)[:, :D]


def _local_ppf_transformer(tp, xq_raw, xg, ppf):
    q = xq_raw @ tp["q"]["w"] + tp["q"]["b"]
    k = xg @ tp["k"]["w"] + tp["k"]["b"]
    v = xg @ tp["v"]["w"] + tp["v"]["b"]
    pe = jax.nn.relu(ppf @ tp["p1"]["w"] + tp["p1"]["b"]) @ tp["p2"]["w"] + tp["p2"]["b"]
    M, K, h = k.shape
    dh = h // NUM_HEADS
    qh = q.reshape(M, NUM_HEADS, dh)
    kh = (k + pe).reshape(M, K, NUM_HEADS, dh)
    vh = (v + pe).reshape(M, K, NUM_HEADS, dh)
    attn = jnp.einsum('mhd,mkhd->mkh', qh, kh) / np.sqrt(dh)
    attn = jax.nn.softmax(attn, axis=1)
    out = jnp.einsum('mkh,mkhd->mhd', attn, vh).reshape(M, h)
    return out @ tp["o"]["w"] + tp["o"]["b"]


def _layer_norm(x, g, b):
    mu = jnp.mean(x, -1, keepdims=True)
    var = jnp.var(x, -1, keepdims=True)
    return (x - mu) / jnp.sqrt(var + 1e-5) * g + b


def _mlp_kernel_body(x_ref, *refs):
    out_ref = refs[-1]
    wrefs = refs[:-1]
    x = x_ref[...]
    nl = len(wrefs) // 2
    for i in range(nl):
        w = wrefs[2 * i][...]
        b = wrefs[2 * i + 1][...]
        x = jnp.dot(x, w, preferred_element_type=jnp.float32) + b[None, :]
        x = jnp.tanh(x) if i == nl - 1 else jax.nn.relu(x)
    out_ref[...] = x


def _mlp_pallas(x, mlp_params):
    M = x.shape[0]
    args = [x]
    for lp in mlp_params:
        args.append(lp["w"])
        args.append(lp["b"])
    out = pl.pallas_call(
        _mlp_kernel_body,
        out_shape=jax.ShapeDtypeStruct((M, MLP_DIMS[-1]), jnp.float32),
    )(*args)
    return out


def _forward(params, p, x, n):
    for s in range(5):
        stage = params["stages"][s]
        stride, nsample = STRIDE[s], NSAMPLE[s]
        N = p.shape[0]
        din = x.shape[1]
        pnx = jnp.concatenate([p, n, x], axis=1)
        if stride != 1:
            m = N // stride
            idx = _fps_idx(p, m)
            q_rows = _gather_rows(pnx, idx)
            n_p = q_rows[:, 0:3]
            n_n = q_rows[:, 3:6]
            xq = q_rows[:, 6:]
        else:
            m = N
            n_p, n_n = p, n
            xq = x
        group_idx = _knn_idx(n_p, p, nsample)
        gg = _gather_rows(pnx, group_idx.reshape(-1)).reshape(m, nsample, -1)
        ppf = _calc_ppf(n_p, n_n, gg[..., 0:3], gg[..., 3:6])
        xg = gg[..., 6:]
        x = _local_ppf_transformer(stage["down"], xq, xg, ppf)
        p, n = n_p, n_n
        if len(stage["blocks"]) > 0:
            if stride == 1:
                bg_idx = group_idx
                bppf = ppf
            else:
                pn = jnp.concatenate([p, n], axis=1)
                bg_idx = _knn_idx(p, p, nsample)
                bgg = _gather_rows(pn, bg_idx.reshape(-1)).reshape(m, nsample, 6)
                bppf = _calc_ppf(p, n, bgg[..., 0:3], bgg[..., 3:6])
            bgf = bg_idx.reshape(-1)
            for blk in stage["blocks"]:
                identity = x
                bxg = _gather_rows(x, bgf).reshape(m, nsample, -1)
                xt = _local_ppf_transformer(blk["tr"], x, bxg, bppf)
                xt = _layer_norm(xt, blk["ln_g"], blk["ln_b"])
                x = jax.nn.relu(xt + identity)
    return _mlp_pallas(x, params["mlp"])


def kernel(p0, x0, o0, n0, params):
    return _forward(params, p0, x0, n0)
